# baseline (speedup 1.0000x reference)
import functools

import jax
import jax.numpy as jnp
from jax import lax
from jax.experimental import pallas as pl
from jax.experimental.pallas import tpu as pltpu
from jax.experimental.pallas import tpu_sc as plsc

N = 50000
E = 800000
STATE_DIM = 16
NUM_AGENTS = 1000

NC = 2
NS = 16
NW = NC * NS

RNODE = 1568
NPAD = NW * RNODE

_MESH = plsc.VectorSubcoreMesh(
    core_axis_name="c", subcore_axis_name="s", num_cores=NC, num_subcores=NS)

F32 = jnp.float32


def _worker_id():
  return lax.axis_index("s") * NC + lax.axis_index("c")


def _gather_body(feat, h_hbm, src_hbm, dst_hbm, outs_hbm, outd_hbm,
                 idx_v, rows_v, sem):
  del feat
  wid = _worker_id()
  epw = E // NW
  chb = 1000
  base = wid * epw

  def one_pass(idx_hbm, out_hbm):
    def chunk(i, carry):
      off = base + i * chb
      pltpu.sync_copy(idx_hbm.at[pl.ds(off, chb)], idx_v)
      descs = []
      for j in range(7):
        descs.append(pltpu.async_copy(
            h_hbm.at[idx_v.at[pl.ds(j * 128, 128)]],
            rows_v.at[pl.ds(j * 128, 128)], sem))
      descs.append(pltpu.async_copy(
          h_hbm.at[idx_v.at[pl.ds(896, 104)]],
          rows_v.at[pl.ds(896, 104)], sem))
      for d in descs:
        d.wait()
      pltpu.sync_copy(rows_v, out_hbm.at[pl.ds(off, chb)])
      return carry
    lax.fori_loop(0, epw // chb, chunk, 0)

  one_pass(src_hbm, outs_hbm)
  one_pass(dst_hbm, outd_hbm)


def _make_gather(feat):
  return pl.kernel(
      functools.partial(_gather_body, feat),
      out_type=(jax.ShapeDtypeStruct((E, feat), F32),
                jax.ShapeDtypeStruct((E, feat), F32)),
      mesh=_MESH,
      scratch_types=[
          pltpu.VMEM((1000,), jnp.int32),
          pltpu.VMEM((1000, feat), F32),
          pltpu.SemaphoreType.DMA,
      ],
      compiler_params=pltpu.CompilerParams(use_tc_tiling_on_sc=False),
      name=f"sc_gather_f{feat}",
  )


_SEG_CH = 4000
_SEG_G = 128
_NEG = -3.0e38
_DBITS = 11


def _segmax_body(msg_hbm, dst_hbm, aggr_hbm, dstbuf, pids, gidx, rows,
                 acc, sem):
  wid = _worker_id()
  lo = wid * RNODE

  def init_row(j, carry):
    for k in range(4):
      acc[j, pl.ds(k * 16, 16)] = jnp.full((16,), _NEG, F32)
    return carry
  lax.fori_loop(0, RNODE, init_row, 0)

  def chunk(c, carry):
    pltpu.sync_copy(dst_hbm.at[pl.ds(c * _SEG_CH, _SEG_CH)], dstbuf)

    def filt(j, cnt):
      d16 = dstbuf[pl.ds(j * 16, 16)]
      dl = d16 - lo
      msk = (dl >= 0) & (dl < RNODE)
      ids = (c * _SEG_CH + j * 16) + lax.iota(jnp.int32, 16)
      packed = (ids << _DBITS) | jnp.where(msk, dl, 0)
      _, sel, _ = plsc.sort_key_val(dl, packed, mask=msk)
      pids[pl.ds(cnt, 16)] = sel
      return cnt + plsc.all_reduce_population_count(msk)[0]
    cnt = lax.fori_loop(0, _SEG_CH // 16, filt, 0)

    for k in range(_SEG_G // 16):
      pids[pl.ds(cnt + k * 16, 16)] = jnp.zeros((16,), jnp.int32)

    ngrp = (cnt + _SEG_G - 1) // _SEG_G

    def grp(g, carry2):
      for k in range(_SEG_G // 16):
        gidx[pl.ds(k * 16, 16)] = (
            pids[pl.ds(g * _SEG_G + k * 16, 16)] >> _DBITS)
      pltpu.async_copy(msg_hbm.at[gidx], rows, sem).wait()
      gn = jnp.minimum(_SEG_G, cnt - g * _SEG_G)

      def upd(r, carry3):
        d = pids[pl.ds(g * _SEG_G + r, 16)][0] & ((1 << _DBITS) - 1)
        for k in range(4):
          sl = pl.ds(k * 16, 16)
          acc[d, sl] = jnp.maximum(acc[d, sl], rows[r, sl])
        return carry3
      lax.fori_loop(0, gn, upd, 0)
      return carry2
    lax.fori_loop(0, ngrp, grp, 0)
    return carry
  lax.fori_loop(0, E // _SEG_CH, chunk, 0)

  def fin_row(j, carry):
    for k in range(4):
      sl = pl.ds(k * 16, 16)
      v = acc[j, sl]
      acc[j, sl] = jnp.where(v > -1.0e38, v, 0.0)
    return carry
  lax.fori_loop(0, RNODE, fin_row, 0)

  pltpu.sync_copy(acc, aggr_hbm.at[pl.ds(lo, RNODE)])


_segmax = pl.kernel(
    _segmax_body,
    out_type=jax.ShapeDtypeStruct((NPAD, 64), F32),
    mesh=_MESH,
    scratch_types=[
        pltpu.VMEM((_SEG_CH,), jnp.int32),
        pltpu.VMEM((_SEG_CH + _SEG_G,), jnp.int32),
        pltpu.VMEM((_SEG_G,), jnp.int32),
        pltpu.VMEM((_SEG_G, 64), F32),
        pltpu.VMEM((RNODE, 64), F32),
        pltpu.SemaphoreType.DMA,
    ],
    compiler_params=pltpu.CompilerParams(
        use_tc_tiling_on_sc=False, needs_layout_passes=False),
    name="sc_segmax",
)


_BE = 2000
_BN = 2000


def _dot(a, b):
  return jnp.dot(a, b, preferred_element_type=F32)


def _phi_kernel(hd, hs, at, wd, ws, wa, b1, w2, b2, w3, b3, out):
  y = _dot(hd[...], wd[...]) + _dot(hs[...], ws[...]) + _dot(at[...], wa[...])
  y = jnp.maximum(y + b1[...], 0.0)
  y = jnp.maximum(_dot(y, w2[...]) + b2[...], 0.0)
  out[...] = _dot(y, w3[...]) + b3[...]


def _full(shape):
  return pl.BlockSpec(shape, lambda i: (0, 0))


def _make_phi(feat):
  return pl.pallas_call(
      _phi_kernel,
      grid=(E // _BE,),
      in_specs=[
          pl.BlockSpec((_BE, feat), lambda i: (i, 0)),
          pl.BlockSpec((_BE, feat), lambda i: (i, 0)),
          pl.BlockSpec((_BE, STATE_DIM), lambda i: (i, 0)),
          _full((feat, 64)), _full((feat, 64)), _full((STATE_DIM, 64)),
          _full((1, 64)), _full((64, 64)), _full((1, 64)),
          _full((64, 64)), _full((1, 64)),
      ],
      out_specs=pl.BlockSpec((_BE, 64), lambda i: (i, 0)),
      out_shape=jax.ShapeDtypeStruct((E, 64), F32),
      name=f"tc_phi_f{feat}",
  )


def _gamma_kernel(final_relu, ag, h, wa, wh, b1, w2, b2, w3, b3, out):
  y = _dot(ag[...], wa[...]) + _dot(h[...], wh[...])
  y = jnp.maximum(y + b1[...], 0.0)
  y = jnp.maximum(_dot(y, w2[...]) + b2[...], 0.0)
  y = _dot(y, w3[...]) + b3[...]
  if final_relu:
    y = jnp.maximum(y, 0.0)
  out[...] = y


def _make_gamma(feat, final_relu):
  return pl.pallas_call(
      functools.partial(_gamma_kernel, final_relu),
      grid=(N // _BN,),
      in_specs=[
          pl.BlockSpec((_BN, 64), lambda i: (i, 0)),
          pl.BlockSpec((_BN, feat), lambda i: (i, 0)),
          _full((64, 64)), _full((feat, 64)), _full((1, 64)),
          _full((64, 64)), _full((1, 64)),
          _full((64, 64)), _full((1, 64)),
      ],
      out_specs=pl.BlockSpec((_BN, 64), lambda i: (i, 0)),
      out_shape=jax.ShapeDtypeStruct((N, 64), F32),
      name=f"tc_gamma_f{feat}",
  )


def _gamma_head_kernel(ag, h, wa, wh, b1, w2, b2, w3, b3,
                       hw1, hb1, hw2, hb2, hw3, hb3, out):
  y = _dot(ag[...], wa[...]) + _dot(h[...], wh[...])
  y = jnp.maximum(y + b1[...], 0.0)
  y = jnp.maximum(_dot(y, w2[...]) + b2[...], 0.0)
  y = _dot(y, w3[...]) + b3[...]
  z = jnp.maximum(_dot(y, hw1[...]) + hb1[...], 0.0)
  z = jnp.maximum(_dot(z, hw2[...]) + hb2[...], 0.0)
  out[...] = _dot(z, hw3[...]) + hb3[...]


_gamma_head = pl.pallas_call(
    _gamma_head_kernel,
    grid=(N // _BN,),
    in_specs=[
        pl.BlockSpec((_BN, 64), lambda i: (i, 0)),
        pl.BlockSpec((_BN, 64), lambda i: (i, 0)),
        _full((64, 64)), _full((64, 64)), _full((1, 64)),
        _full((64, 64)), _full((1, 64)),
        _full((64, 64)), _full((1, 64)),
        _full((64, 64)), _full((1, 64)),
        _full((64, 64)), _full((1, 64)),
        _full((64, 1)), _full((1, 1)),
    ],
    out_specs=pl.BlockSpec((_BN, 1), lambda i: (i, 0)),
    out_shape=jax.ShapeDtypeStruct((N, 1), F32),
    name="tc_gamma_head",
)

_gather16 = _make_gather(16)
_gather64 = _make_gather(64)
_phi16 = _make_phi(16)
_phi64 = _make_phi(64)
_gamma16 = _make_gamma(16, True)


def _split_phi_w(p, feat):
  (w1, b1), (w2, b2), (w3, b3) = p
  wd = w1[:feat]
  ws = w1[feat:2 * feat]
  wa = w1[2 * feat:]
  return (wd, ws, wa, b1.reshape(1, 64), w2, b2.reshape(1, 64),
          w3, b3.reshape(1, 64))


def _split_gamma_w(p):
  (w1, b1), (w2, b2), (w3, b3) = p
  wa = w1[:64]
  wh = w1[64:]
  return (wa, wh, b1.reshape(1, 64), w2, b2.reshape(1, 64),
          w3, b3.reshape(1, 64))


def kernel(x, edge_attr, edge_index, params):
  src = edge_index[0]
  dst = edge_index[1]

  hs1, hd1 = _gather16(x, src, dst)
  msg1 = _phi16(hd1, hs1, edge_attr, *_split_phi_w(params['phi1'], 16))
  ag1 = _segmax(msg1, dst)
  h1 = _gamma16(ag1[:N], x, *_split_gamma_w(params['gamma1']))

  hs2, hd2 = _gather64(h1, src, dst)
  msg2 = _phi64(hd2, hs2, edge_attr, *_split_phi_w(params['phi2'], 64))
  ag2 = _segmax(msg2, dst)

  (hw1, hb1), (hw2, hb2), (hw3, hb3) = params['head']
  out = _gamma_head(ag2[:N], h1, *_split_gamma_w(params['gamma2']),
                    hw1, hb1.reshape(1, 64), hw2, hb2.reshape(1, 64),
                    hw3, hb3.reshape(1, 1))
  return out.reshape(-1, NUM_AGENTS)


# speedup vs baseline: 1.2273x; 1.2273x over previous
import functools

import jax
import jax.numpy as jnp
from jax import lax
from jax.experimental import pallas as pl
from jax.experimental.pallas import tpu as pltpu
from jax.experimental.pallas import tpu_sc as plsc

N = 50000
E = 800000
STATE_DIM = 16
NUM_AGENTS = 1000

NC = 2
NS = 16
NW = NC * NS

RNODE = 1568
NPAD = NW * RNODE

_MESH = plsc.VectorSubcoreMesh(
    core_axis_name="c", subcore_axis_name="s", num_cores=NC, num_subcores=NS)

F32 = jnp.float32


def _worker_id():
  return lax.axis_index("s") * NC + lax.axis_index("c")


def _gather_body(feat, h_hbm, src_hbm, dst_hbm, outs_hbm, outd_hbm,
                 idx_v, rows_v, sem):
  del feat
  wid = _worker_id()
  epw = E // NW
  chb = 1000
  base = wid * epw

  def one_pass(idx_hbm, out_hbm):
    def chunk(i, carry):
      off = base + i * chb
      pltpu.sync_copy(idx_hbm.at[pl.ds(off, chb)], idx_v)
      descs = []
      for j in range(7):
        descs.append(pltpu.async_copy(
            h_hbm.at[idx_v.at[pl.ds(j * 128, 128)]],
            rows_v.at[pl.ds(j * 128, 128)], sem))
      descs.append(pltpu.async_copy(
          h_hbm.at[idx_v.at[pl.ds(896, 104)]],
          rows_v.at[pl.ds(896, 104)], sem))
      for d in descs:
        d.wait()
      pltpu.sync_copy(rows_v, out_hbm.at[pl.ds(off, chb)])
      return carry
    lax.fori_loop(0, epw // chb, chunk, 0)

  one_pass(src_hbm, outs_hbm)
  one_pass(dst_hbm, outd_hbm)


def _make_gather(feat):
  return pl.kernel(
      functools.partial(_gather_body, feat),
      out_type=(jax.ShapeDtypeStruct((E, feat), F32),
                jax.ShapeDtypeStruct((E, feat), F32)),
      mesh=_MESH,
      scratch_types=[
          pltpu.VMEM((1000,), jnp.int32),
          pltpu.VMEM((1000, feat), F32),
          pltpu.SemaphoreType.DMA,
      ],
      compiler_params=pltpu.CompilerParams(use_tc_tiling_on_sc=False),
      name=f"sc_gather_f{feat}",
  )


_SEG_CH = 4000
_NCHUNK = E // _SEG_CH
_NEG = -3.0e38


def _segmax_body(msgT_hbm, dst_hbm, aggrT_hbm, dstb, mc, acc,
                 sd0, sd1, sm0, sm1):
  wid = _worker_id()
  f0 = wid * 2
  dsems = (sd0, sd1)
  msems = (sm0, sm1)

  def initf(j, carry):
    acc[pl.ds(j * 16, 16)] = jnp.full((16,), _NEG, F32)
    return carry
  lax.fori_loop(0, (2 * N) // 16, initf, 0)

  def issue(c, buf):
    pltpu.async_copy(dst_hbm.at[pl.ds(c * _SEG_CH, _SEG_CH)],
                     dstb.at[buf], dsems[buf])
    pltpu.async_copy(msgT_hbm.at[pl.ds(f0, 2), pl.ds(c * _SEG_CH, _SEG_CH)],
                     mc.at[buf], msems[buf])

  def wait(buf):
    pltpu.make_async_copy(dst_hbm.at[pl.ds(0, _SEG_CH)],
                          dstb.at[buf], dsems[buf]).wait()
    pltpu.make_async_copy(msgT_hbm.at[pl.ds(0, 2), pl.ds(0, _SEG_CH)],
                          mc.at[buf], msems[buf]).wait()

  issue(0, 0)
  issue(1, 1)

  def win(buf):
    def body(w, carry):
      sl = pl.ds(w * 16, 16)
      d16 = dstb[buf, sl]
      i1 = d16 + N
      m0 = mc[buf, 0, sl]
      m1 = mc[buf, 1, sl]
      g0 = plsc.load_gather(acc, [d16])
      g1 = plsc.load_gather(acc, [i1])
      n0 = jnp.maximum(g0, m0)
      n1 = jnp.maximum(g1, m1)
      plsc.store_scatter(acc, [d16], n0)
      plsc.store_scatter(acc, [i1], n1)
      r0 = plsc.load_gather(acc, [d16])
      r1 = plsc.load_gather(acc, [i1])
      lost0 = r0 < n0
      lost1 = r1 < n1
      nl = plsc.all_reduce_population_count(lost0 | lost1)[0]

      @pl.when(nl > 0)
      def _fix():
        def cond(st):
          return st[2] > 0

        def fbody(st):
          l0, l1, _ = st
          plsc.store_scatter(acc, [d16], n0, mask=l0)
          plsc.store_scatter(acc, [i1], n1, mask=l1)
          q0 = plsc.load_gather(acc, [d16])
          q1 = plsc.load_gather(acc, [i1])
          l0n = (q0 < n0) & l0
          l1n = (q1 < n1) & l1
          return (l0n, l1n,
                  plsc.all_reduce_population_count(l0n | l1n)[0])
        lax.while_loop(cond, fbody, (lost0, lost1, nl))
      return carry
    return body

  def chunk_pair(c2, carry):
    for buf in (0, 1):
      c = c2 * 2 + buf
      wait(buf)
      lax.fori_loop(0, _SEG_CH // 16, win(buf), 0)

      @pl.when(c + 2 < _NCHUNK)
      def _pref():
        issue(c + 2, buf)
    return carry
  lax.fori_loop(0, _NCHUNK // 2, chunk_pair, 0)

  def finf(j, carry):
    sl = pl.ds(j * 16, 16)
    v = acc[sl]
    acc[sl] = jnp.where(v > -1.0e38, v, 0.0)
    return carry
  lax.fori_loop(0, (2 * N) // 16, finf, 0)

  pltpu.sync_copy(acc.at[pl.ds(0, N)], aggrT_hbm.at[f0])
  pltpu.sync_copy(acc.at[pl.ds(N, N)], aggrT_hbm.at[f0 + 1])


_segmax = pl.kernel(
    _segmax_body,
    out_type=jax.ShapeDtypeStruct((64, N), F32),
    mesh=_MESH,
    scratch_types=[
        pltpu.VMEM((2, _SEG_CH), jnp.int32),
        pltpu.VMEM((2, 2, _SEG_CH), F32),
        pltpu.VMEM((2 * N,), F32),
        pltpu.SemaphoreType.DMA,
        pltpu.SemaphoreType.DMA,
        pltpu.SemaphoreType.DMA,
        pltpu.SemaphoreType.DMA,
    ],
    compiler_params=pltpu.CompilerParams(
        use_tc_tiling_on_sc=False, needs_layout_passes=False),
    name="sc_segmax",
)


_BE = 3200
_BN = 2048
_GN = -(-N // _BN)


def _dot(a, b):
  return jnp.dot(a, b, preferred_element_type=F32)


def _phi_kernel(hd, hs, at, wd, ws, wa, b1, w2, b2, w3, b3, out):
  y = _dot(hd[...], wd[...]) + _dot(hs[...], ws[...]) + _dot(at[...], wa[...])
  y = jnp.maximum(y + b1[...], 0.0)
  y = jnp.maximum(_dot(y, w2[...]) + b2[...], 0.0)
  out[...] = lax.dot_general(
      w3[...], y, (((0,), (1,)), ((), ())),
      preferred_element_type=F32) + b3[...]


def _full(shape):
  return pl.BlockSpec(shape, lambda i: (0, 0))


def _make_phi(feat):
  return pl.pallas_call(
      _phi_kernel,
      grid=(E // _BE,),
      in_specs=[
          pl.BlockSpec((_BE, feat), lambda i: (i, 0)),
          pl.BlockSpec((_BE, feat), lambda i: (i, 0)),
          pl.BlockSpec((_BE, STATE_DIM), lambda i: (i, 0)),
          _full((feat, 64)), _full((feat, 64)), _full((STATE_DIM, 64)),
          _full((1, 64)), _full((64, 64)), _full((1, 64)),
          _full((64, 64)), _full((64, 1)),
      ],
      out_specs=pl.BlockSpec((64, _BE), lambda i: (0, i)),
      out_shape=jax.ShapeDtypeStruct((64, E), F32),
      name=f"tc_phi_f{feat}",
  )


def _gamma_kernel(final_relu, ag, h, wa, wh, b1, w2, b2, w3, b3, out):
  y = lax.dot_general(ag[...], wa[...], (((0,), (0,)), ((), ())),
                      preferred_element_type=F32) + _dot(h[...], wh[...])
  y = jnp.maximum(y + b1[...], 0.0)
  y = jnp.maximum(_dot(y, w2[...]) + b2[...], 0.0)
  y = _dot(y, w3[...]) + b3[...]
  if final_relu:
    y = jnp.maximum(y, 0.0)
  out[...] = y


def _make_gamma(feat, final_relu):
  return pl.pallas_call(
      functools.partial(_gamma_kernel, final_relu),
      grid=(_GN,),
      in_specs=[
          pl.BlockSpec((64, _BN), lambda i: (0, i)),
          pl.BlockSpec((_BN, feat), lambda i: (i, 0)),
          _full((64, 64)), _full((feat, 64)), _full((1, 64)),
          _full((64, 64)), _full((1, 64)),
          _full((64, 64)), _full((1, 64)),
      ],
      out_specs=pl.BlockSpec((_BN, 64), lambda i: (i, 0)),
      out_shape=jax.ShapeDtypeStruct((N, 64), F32),
      name=f"tc_gamma_f{feat}",
  )


def _gamma_head_kernel(ag, h, wa, wh, b1, w2, b2, w3, b3,
                       hw1, hb1, hw2, hb2, hw3, hb3, out):
  y = lax.dot_general(ag[...], wa[...], (((0,), (0,)), ((), ())),
                      preferred_element_type=F32) + _dot(h[...], wh[...])
  y = jnp.maximum(y + b1[...], 0.0)
  y = jnp.maximum(_dot(y, w2[...]) + b2[...], 0.0)
  y = _dot(y, w3[...]) + b3[...]
  z = jnp.maximum(_dot(y, hw1[...]) + hb1[...], 0.0)
  z = jnp.maximum(_dot(z, hw2[...]) + hb2[...], 0.0)
  out[...] = _dot(z, hw3[...]) + hb3[...]


_gamma_head = pl.pallas_call(
    _gamma_head_kernel,
    grid=(_GN,),
    in_specs=[
        pl.BlockSpec((64, _BN), lambda i: (0, i)),
        pl.BlockSpec((_BN, 64), lambda i: (i, 0)),
        _full((64, 64)), _full((64, 64)), _full((1, 64)),
        _full((64, 64)), _full((1, 64)),
        _full((64, 64)), _full((1, 64)),
        _full((64, 64)), _full((1, 64)),
        _full((64, 64)), _full((1, 64)),
        _full((64, 1)), _full((1, 1)),
    ],
    out_specs=pl.BlockSpec((_BN, 1), lambda i: (i, 0)),
    out_shape=jax.ShapeDtypeStruct((N, 1), F32),
    name="tc_gamma_head",
)

_gather16 = _make_gather(16)
_gather64 = _make_gather(64)
_phi16 = _make_phi(16)
_phi64 = _make_phi(64)
_gamma16 = _make_gamma(16, True)


def _split_phi_w(p, feat):
  (w1, b1), (w2, b2), (w3, b3) = p
  wd = w1[:feat]
  ws = w1[feat:2 * feat]
  wa = w1[2 * feat:]
  return (wd, ws, wa, b1.reshape(1, 64), w2, b2.reshape(1, 64),
          w3, b3.reshape(64, 1))


def _split_gamma_w(p):
  (w1, b1), (w2, b2), (w3, b3) = p
  wa = w1[:64]
  wh = w1[64:]
  return (wa, wh, b1.reshape(1, 64), w2, b2.reshape(1, 64),
          w3, b3.reshape(1, 64))


def kernel(x, edge_attr, edge_index, params):
  src = edge_index[0]
  dst = edge_index[1]

  hs1, hd1 = _gather16(x, src, dst)
  msg1 = _phi16(hd1, hs1, edge_attr, *_split_phi_w(params['phi1'], 16))
  ag1 = _segmax(msg1, dst)
  h1 = _gamma16(ag1, x, *_split_gamma_w(params['gamma1']))

  hs2, hd2 = _gather64(h1, src, dst)
  msg2 = _phi64(hd2, hs2, edge_attr, *_split_phi_w(params['phi2'], 64))
  ag2 = _segmax(msg2, dst)

  (hw1, hb1), (hw2, hb2), (hw3, hb3) = params['head']
  out = _gamma_head(ag2, h1, *_split_gamma_w(params['gamma2']),
                    hw1, hb1.reshape(1, 64), hw2, hb2.reshape(1, 64),
                    hw3, hb3.reshape(1, 1))
  return out.reshape(-1, NUM_AGENTS)


# speedup vs baseline: 2.6578x; 2.1656x over previous
import functools

import jax
import jax.numpy as jnp
from jax import lax
from jax.experimental import pallas as pl
from jax.experimental.pallas import tpu as pltpu
from jax.experimental.pallas import tpu_sc as plsc

N = 50000
E = 800000
STATE_DIM = 16
NUM_AGENTS = 1000
FP = 128
NROW = E // FP
NPAD = 392

NC = 2
NS = 16
NW = NC * NS

_MESH = plsc.VectorSubcoreMesh(
    core_axis_name="c", subcore_axis_name="s", num_cores=NC, num_subcores=NS)

F32 = jnp.float32


def _worker_id():
  return lax.axis_index("s") * NC + lax.axis_index("c")


def _gather_body(h_hbm, src_hbm, dst_hbm, outs_hbm, outd_hbm,
                 idx_v, rows_v, sem):
  wid = _worker_id()
  epw = E // NW
  chb = 1000
  base = wid * epw

  def one_pass(idx_hbm, out_hbm):
    def chunk(i, carry):
      off = base + i * chb
      pltpu.sync_copy(idx_hbm.at[pl.ds(off, chb)], idx_v)
      descs = []
      for j in range(7):
        descs.append(pltpu.async_copy(
            h_hbm.at[idx_v.at[pl.ds(j * 128, 128)]],
            rows_v.at[pl.ds(j * 128, 128)], sem))
      descs.append(pltpu.async_copy(
          h_hbm.at[idx_v.at[pl.ds(896, 104)]],
          rows_v.at[pl.ds(896, 104)], sem))
      for d in descs:
        d.wait()
      pltpu.sync_copy(rows_v, out_hbm.at[pl.ds(off, chb)])
      return carry
    lax.fori_loop(0, epw // chb, chunk, 0)

  one_pass(src_hbm, outs_hbm)
  one_pass(dst_hbm, outd_hbm)


_gather = pl.kernel(
    _gather_body,
    out_type=(jax.ShapeDtypeStruct((E, FP), F32),
              jax.ShapeDtypeStruct((E, FP), F32)),
    mesh=_MESH,
    scratch_types=[
        pltpu.VMEM((1000,), jnp.int32),
        pltpu.VMEM((1000, FP), F32),
        pltpu.SemaphoreType.DMA,
    ],
    compiler_params=pltpu.CompilerParams(use_tc_tiling_on_sc=False),
    name="sc_gather",
)


_SEG_CH = 3200
_SEG_ROWS = _SEG_CH // 128
_NCHUNK = E // _SEG_CH
_NEG = -3.0e38


def _segmax_body(msgT_hbm, dst_hbm, aggrT_hbm, dstb, mc, acc,
                 sd0, sd1, sm0, sm1):
  wid = _worker_id()
  f0 = wid * 2
  dsems = (sd0, sd1)
  msems = (sm0, sm1)

  def init_row(r, carry):
    for f in range(2):
      for k in range(8):
        acc[f, r, pl.ds(k * 16, 16)] = jnp.full((16,), _NEG, F32)
    return carry
  lax.fori_loop(0, NPAD, init_row, 0)

  def issue(c, buf):
    pltpu.async_copy(dst_hbm.at[pl.ds(c * _SEG_CH, _SEG_CH)],
                     dstb.at[buf], dsems[buf])
    pltpu.async_copy(
        msgT_hbm.at[pl.ds(c * _SEG_ROWS, _SEG_ROWS), pl.ds(f0, 2)],
        mc.at[buf], msems[buf])

  def wait(buf):
    pltpu.make_async_copy(dst_hbm.at[pl.ds(0, _SEG_CH)],
                          dstb.at[buf], dsems[buf]).wait()
    pltpu.make_async_copy(msgT_hbm.at[pl.ds(0, _SEG_ROWS), pl.ds(0, 2)],
                          mc.at[buf], msems[buf]).wait()

  issue(0, 0)
  issue(1, 1)

  zeros16 = jnp.zeros((16,), jnp.int32)
  ones16 = jnp.ones((16,), jnp.int32)

  def win(buf):
    def body(w, carry):
      whi = w // 8
      wlo = (w % 8) * 16
      d16 = dstb[buf, pl.ds(w * 16, 16)]
      dhi = d16 >> 7
      dlo = d16 & 127
      i0 = [zeros16, dhi, dlo]
      i1 = [ones16, dhi, dlo]
      m0 = mc[buf, whi, 0, pl.ds(wlo, 16)]
      m1 = mc[buf, whi, 1, pl.ds(wlo, 16)]
      g0 = plsc.load_gather(acc, i0)
      g1 = plsc.load_gather(acc, i1)
      n0 = jnp.maximum(g0, m0)
      n1 = jnp.maximum(g1, m1)
      plsc.store_scatter(acc, i0, n0)
      plsc.store_scatter(acc, i1, n1)
      r0 = plsc.load_gather(acc, i0)
      r1 = plsc.load_gather(acc, i1)
      lost0 = r0 < n0
      lost1 = r1 < n1
      nl = plsc.all_reduce_population_count(lost0 | lost1)[0]

      @pl.when(nl > 0)
      def _fix():
        def cond(st):
          return st[2] > 0

        def fbody(st):
          l0, l1, _ = st
          plsc.store_scatter(acc, i0, n0, mask=l0)
          plsc.store_scatter(acc, i1, n1, mask=l1)
          q0 = plsc.load_gather(acc, i0)
          q1 = plsc.load_gather(acc, i1)
          l0n = (q0 < n0) & l0
          l1n = (q1 < n1) & l1
          return (l0n, l1n,
                  plsc.all_reduce_population_count(l0n | l1n)[0])
        lax.while_loop(cond, fbody, (lost0, lost1, nl))
      return carry
    return body

  def chunk_pair(c2, carry):
    for buf in (0, 1):
      c = c2 * 2 + buf
      wait(buf)
      lax.fori_loop(0, _SEG_CH // 16, win(buf), 0)

      @pl.when(c + 2 < _NCHUNK)
      def _pref():
        issue(c + 2, buf)
    return carry
  lax.fori_loop(0, _NCHUNK // 2, chunk_pair, 0)

  def fin_row(r, carry):
    for f in range(2):
      for k in range(8):
        sl = pl.ds(k * 16, 16)
        v = acc[f, r, sl]
        acc[f, r, sl] = jnp.where(v > -1.0e38, v, 0.0)
    return carry
  lax.fori_loop(0, NPAD, fin_row, 0)

  pltpu.sync_copy(acc, aggrT_hbm.at[pl.ds(f0, 2)])


_segmax = pl.kernel(
    _segmax_body,
    out_type=jax.ShapeDtypeStruct((64, NPAD, 128), F32),
    mesh=_MESH,
    scratch_types=[
        pltpu.VMEM((2, _SEG_CH), jnp.int32),
        pltpu.VMEM((2, _SEG_ROWS, 2, 128), F32),
        pltpu.VMEM((2, NPAD, 128), F32),
        pltpu.SemaphoreType.DMA,
        pltpu.SemaphoreType.DMA,
        pltpu.SemaphoreType.DMA,
        pltpu.SemaphoreType.DMA,
    ],
    compiler_params=pltpu.CompilerParams(
        use_tc_tiling_on_sc=False, needs_layout_passes=False),
    name="sc_segmax",
)


_BE = 3200
_BN = 2048
_GN = -(-N // _BN)


def _dot(a, b):
  return jnp.dot(a, b, preferred_element_type=F32)


def _phi_kernel(hd, hs, at, wd, ws, wa, b1, w2, b2, w3, b3, out):
  y = _dot(hd[...], wd[...]) + _dot(hs[...], ws[...]) + _dot(at[...], wa[...])
  y = jnp.maximum(y + b1[...], 0.0)
  y = jnp.maximum(_dot(y, w2[...]) + b2[...], 0.0)
  yt = lax.dot_general(
      w3[...], y, (((0,), (1,)), ((), ())),
      preferred_element_type=F32) + b3[...]
  for r in range(_BE // 128):
    out[r] = yt[:, r * 128:(r + 1) * 128]


def _full(shape):
  idx = lambda i: tuple(0 for _ in shape)
  return pl.BlockSpec(shape, idx)


_phi = pl.pallas_call(
    _phi_kernel,
    grid=(E // _BE,),
    in_specs=[
        pl.BlockSpec((_BE, FP), lambda i: (i, 0)),
        pl.BlockSpec((_BE, FP), lambda i: (i, 0)),
        pl.BlockSpec((_BE, STATE_DIM), lambda i: (i, 0)),
        _full((FP, 64)), _full((FP, 64)), _full((STATE_DIM, 64)),
        _full((1, 64)), _full((64, 64)), _full((1, 64)),
        _full((64, 64)), _full((64, 1)),
    ],
    out_specs=pl.BlockSpec((_BE // 128, 64, 128), lambda i: (i, 0, 0)),
    out_shape=jax.ShapeDtypeStruct((NROW, 64, 128), F32),
    name="tc_phi",
)


def _gamma_kernel(ag, h, wa, wh, b1, w2, b2, w3, b3, out):
  agf = ag[...].reshape(64, _BN)
  y = lax.dot_general(agf, wa[...], (((0,), (0,)), ((), ())),
                      preferred_element_type=F32) + _dot(h[...], wh[...])
  y = jnp.maximum(y + b1[...], 0.0)
  y = jnp.maximum(_dot(y, w2[...]) + b2[...], 0.0)
  y = _dot(y, w3[...]) + b3[...]
  y = jnp.maximum(y, 0.0)
  out[...] = jnp.concatenate([y, jnp.zeros((_BN, FP - 64), F32)], axis=1)


_gamma1 = pl.pallas_call(
    _gamma_kernel,
    grid=(_GN,),
    in_specs=[
        pl.BlockSpec((64, _BN // 128, 128), lambda i: (0, i, 0)),
        pl.BlockSpec((_BN, FP), lambda i: (i, 0)),
        _full((64, 64)), _full((FP, 64)), _full((1, 64)),
        _full((64, 64)), _full((1, 64)),
        _full((64, 64)), _full((1, 64)),
    ],
    out_specs=pl.BlockSpec((_BN, FP), lambda i: (i, 0)),
    out_shape=jax.ShapeDtypeStruct((N, FP), F32),
    name="tc_gamma1",
)


def _gamma_head_kernel(ag, h, wa, wh, b1, w2, b2, w3, b3,
                       hw1, hb1, hw2, hb2, hw3, hb3, out):
  agf = ag[...].reshape(64, _BN)
  y = lax.dot_general(agf, wa[...], (((0,), (0,)), ((), ())),
                      preferred_element_type=F32) + _dot(h[...], wh[...])
  y = jnp.maximum(y + b1[...], 0.0)
  y = jnp.maximum(_dot(y, w2[...]) + b2[...], 0.0)
  y = _dot(y, w3[...]) + b3[...]
  z = jnp.maximum(_dot(y, hw1[...]) + hb1[...], 0.0)
  z = jnp.maximum(_dot(z, hw2[...]) + hb2[...], 0.0)
  out[...] = _dot(z, hw3[...]) + hb3[...]


_gamma_head = pl.pallas_call(
    _gamma_head_kernel,
    grid=(_GN,),
    in_specs=[
        pl.BlockSpec((64, _BN // 128, 128), lambda i: (0, i, 0)),
        pl.BlockSpec((_BN, FP), lambda i: (i, 0)),
        _full((64, 64)), _full((FP, 64)), _full((1, 64)),
        _full((64, 64)), _full((1, 64)),
        _full((64, 64)), _full((1, 64)),
        _full((64, 64)), _full((1, 64)),
        _full((64, 64)), _full((1, 64)),
        _full((64, 1)), _full((1, 1)),
    ],
    out_specs=pl.BlockSpec((_BN, 1), lambda i: (i, 0)),
    out_shape=jax.ShapeDtypeStruct((N, 1), F32),
    name="tc_gamma_head",
)


def _pad_rows(w):
  return jnp.concatenate(
      [w, jnp.zeros((FP - w.shape[0], w.shape[1]), F32)], axis=0)


def _split_phi_w(p, feat):
  (w1, b1), (w2, b2), (w3, b3) = p
  wd = _pad_rows(w1[:feat])
  ws = _pad_rows(w1[feat:2 * feat])
  wa = w1[2 * feat:]
  return (wd, ws, wa, b1.reshape(1, 64), w2, b2.reshape(1, 64),
          w3, b3.reshape(64, 1))


def _split_gamma_w(p):
  (w1, b1), (w2, b2), (w3, b3) = p
  wa = w1[:64]
  wh = _pad_rows(w1[64:])
  return (wa, wh, b1.reshape(1, 64), w2, b2.reshape(1, 64),
          w3, b3.reshape(1, 64))


def kernel(x, edge_attr, edge_index, params):
  src = edge_index[0]
  dst = edge_index[1]
  xp = jnp.concatenate([x, jnp.zeros((N, FP - STATE_DIM), F32)], axis=1)

  hs1, hd1 = _gather(xp, src, dst)
  msg1 = _phi(hd1, hs1, edge_attr, *_split_phi_w(params['phi1'], STATE_DIM))
  ag1 = _segmax(msg1, dst)
  h1 = _gamma1(ag1, xp, *_split_gamma_w(params['gamma1']))

  hs2, hd2 = _gather(h1, src, dst)
  msg2 = _phi(hd2, hs2, edge_attr, *_split_phi_w(params['phi2'], 64))
  ag2 = _segmax(msg2, dst)

  (hw1, hb1), (hw2, hb2), (hw3, hb3) = params['head']
  out = _gamma_head(ag2, h1, *_split_gamma_w(params['gamma2']),
                    hw1, hb1.reshape(1, 64), hw2, hb2.reshape(1, 64),
                    hw3, hb3.reshape(1, 1))
  return out.reshape(-1, NUM_AGENTS)


# speedup vs baseline: 3.9901x; 1.5013x over previous
import functools

import jax
import jax.numpy as jnp
from jax import lax
from jax.experimental import pallas as pl
from jax.experimental.pallas import tpu as pltpu
from jax.experimental.pallas import tpu_sc as plsc

N = 50000
E = 800000
STATE_DIM = 16
NUM_AGENTS = 1000
FP = 128
NROW = E // FP
NPAD = 392

NC = 2
NS = 16
NW = NC * NS

_MESH = plsc.VectorSubcoreMesh(
    core_axis_name="c", subcore_axis_name="s", num_cores=NC, num_subcores=NS)

F32 = jnp.float32


def _worker_id():
  return lax.axis_index("s") * NC + lax.axis_index("c")


def _gather_body(h_hbm, src_hbm, dst_hbm, outs_hbm, outd_hbm,
                 idx_v, rows_v, sem):
  wid = _worker_id()
  epw = E // NW
  chb = 1000
  base = wid * epw

  def one_pass(idx_hbm, out_hbm):
    def chunk(i, carry):
      off = base + i * chb
      pltpu.sync_copy(idx_hbm.at[pl.ds(off, chb)], idx_v)
      descs = []
      for j in range(7):
        descs.append(pltpu.async_copy(
            h_hbm.at[idx_v.at[pl.ds(j * 128, 128)]],
            rows_v.at[pl.ds(j * 128, 128)], sem))
      descs.append(pltpu.async_copy(
          h_hbm.at[idx_v.at[pl.ds(896, 104)]],
          rows_v.at[pl.ds(896, 104)], sem))
      for d in descs:
        d.wait()
      pltpu.sync_copy(rows_v, out_hbm.at[pl.ds(off, chb)])
      return carry
    lax.fori_loop(0, epw // chb, chunk, 0)

  one_pass(src_hbm, outs_hbm)
  one_pass(dst_hbm, outd_hbm)


_gather = pl.kernel(
    _gather_body,
    out_type=(jax.ShapeDtypeStruct((E, FP), F32),
              jax.ShapeDtypeStruct((E, FP), F32)),
    mesh=_MESH,
    scratch_types=[
        pltpu.VMEM((1000,), jnp.int32),
        pltpu.VMEM((1000, FP), F32),
        pltpu.SemaphoreType.DMA,
    ],
    compiler_params=pltpu.CompilerParams(use_tc_tiling_on_sc=False),
    name="sc_gather",
)


_SEG_CH = 3200
_SEG_ROWS = _SEG_CH // 128
_NCHUNK = E // _SEG_CH
_NEG = -3.0e38


def _segmax_body(msgT_hbm, dst_hbm, aggrT_hbm, dstb, mc, acc,
                 sd0, sd1, sm0, sm1):
  wid = _worker_id()
  f0 = wid * 2
  dsems = (sd0, sd1)
  msems = (sm0, sm1)

  def init_row(r, carry):
    for f in range(2):
      for k in range(8):
        acc[f, r, pl.ds(k * 16, 16)] = jnp.full((16,), _NEG, F32)
    return carry
  lax.fori_loop(0, NPAD, init_row, 0)

  def issue(c, buf):
    pltpu.async_copy(dst_hbm.at[pl.ds(c * _SEG_CH, _SEG_CH)],
                     dstb.at[buf], dsems[buf])
    pltpu.async_copy(
        msgT_hbm.at[pl.ds(c * _SEG_ROWS, _SEG_ROWS), pl.ds(f0, 2)],
        mc.at[buf], msems[buf])

  def wait(buf):
    pltpu.make_async_copy(dst_hbm.at[pl.ds(0, _SEG_CH)],
                          dstb.at[buf], dsems[buf]).wait()
    pltpu.make_async_copy(msgT_hbm.at[pl.ds(0, _SEG_ROWS), pl.ds(0, 2)],
                          mc.at[buf], msems[buf]).wait()

  issue(0, 0)
  issue(1, 1)

  zeros16 = jnp.zeros((16,), jnp.int32)
  ones16 = jnp.ones((16,), jnp.int32)
  false16 = jnp.zeros((16,), jnp.bool_)

  def window_indices(buf, rw, sub):
    d16 = dstb[buf, pl.ds(rw * 128 + sub * 16, 16)]
    dhi = d16 >> 7
    dlo = d16 & 127
    return [zeros16, dhi, dlo], [ones16, dhi, dlo]

  def safe_update(buf, rw, sub):
    i0, i1 = window_indices(buf, rw, sub)
    m0 = mc[buf, rw, 0, pl.ds(sub * 16, 16)]
    m1 = mc[buf, rw, 1, pl.ds(sub * 16, 16)]
    n0 = jnp.maximum(plsc.load_gather(acc, i0), m0)
    n1 = jnp.maximum(plsc.load_gather(acc, i1), m1)

    def cond(st):
      return st[2] > 0

    def fbody(st):
      l0, l1, _ = st
      plsc.store_scatter(acc, i0, n0, mask=l0)
      plsc.store_scatter(acc, i1, n1, mask=l1)
      l0n = (plsc.load_gather(acc, i0) < n0) & l0
      l1n = (plsc.load_gather(acc, i1) < n1) & l1
      return (l0n, l1n, plsc.all_reduce_population_count(l0n | l1n)[0])
    lax.while_loop(cond, fbody,
                   (~false16, ~false16, jnp.int32(16)))

  def row_body(buf):
    def body(rw, carry):
      lostacc = false16
      for sub in range(8):
        i0, i1 = window_indices(buf, rw, sub)
        m0 = mc[buf, rw, 0, pl.ds(sub * 16, 16)]
        m1 = mc[buf, rw, 1, pl.ds(sub * 16, 16)]
        n0 = jnp.maximum(plsc.load_gather(acc, i0), m0)
        n1 = jnp.maximum(plsc.load_gather(acc, i1), m1)
        plsc.store_scatter(acc, i0, n0)
        plsc.store_scatter(acc, i1, n1)
        lost0 = plsc.load_gather(acc, i0) < n0
        lost1 = plsc.load_gather(acc, i1) < n1
        lostacc = lostacc | lost0 | lost1
      nl = plsc.all_reduce_population_count(lostacc)[0]

      @pl.when(nl > 0)
      def _fix():
        for sub in range(8):
          safe_update(buf, rw, sub)
      return carry
    return body

  def chunk_pair(c2, carry):
    for buf in (0, 1):
      c = c2 * 2 + buf
      wait(buf)
      lax.fori_loop(0, _SEG_ROWS, row_body(buf), 0)

      @pl.when(c + 2 < _NCHUNK)
      def _pref():
        issue(c + 2, buf)
    return carry
  lax.fori_loop(0, _NCHUNK // 2, chunk_pair, 0)

  def fin_row(r, carry):
    for f in range(2):
      for k in range(8):
        sl = pl.ds(k * 16, 16)
        v = acc[f, r, sl]
        acc[f, r, sl] = jnp.where(v > -1.0e38, v, 0.0)
    return carry
  lax.fori_loop(0, NPAD, fin_row, 0)

  pltpu.sync_copy(acc, aggrT_hbm.at[pl.ds(f0, 2)])


_segmax = pl.kernel(
    _segmax_body,
    out_type=jax.ShapeDtypeStruct((64, NPAD, 128), F32),
    mesh=_MESH,
    scratch_types=[
        pltpu.VMEM((2, _SEG_CH), jnp.int32),
        pltpu.VMEM((2, _SEG_ROWS, 2, 128), F32),
        pltpu.VMEM((2, NPAD, 128), F32),
        pltpu.SemaphoreType.DMA,
        pltpu.SemaphoreType.DMA,
        pltpu.SemaphoreType.DMA,
        pltpu.SemaphoreType.DMA,
    ],
    compiler_params=pltpu.CompilerParams(
        use_tc_tiling_on_sc=False, needs_layout_passes=False),
    name="sc_segmax",
)


_BE = 3200
_BN = 2048
_GN = -(-N // _BN)


def _dot(a, b):
  return jnp.dot(a, b, preferred_element_type=F32)


def _phi_kernel(hd, hs, at, wd, ws, wa, b1, w2, b2, w3, b3, out):
  y = _dot(hd[...], wd[...]) + _dot(hs[...], ws[...]) + _dot(at[...], wa[...])
  y = jnp.maximum(y + b1[...], 0.0)
  y = jnp.maximum(_dot(y, w2[...]) + b2[...], 0.0)
  yt = lax.dot_general(
      w3[...], y, (((0,), (1,)), ((), ())),
      preferred_element_type=F32) + b3[...]
  for r in range(_BE // 128):
    out[r] = yt[:, r * 128:(r + 1) * 128]


def _full(shape):
  idx = lambda i: tuple(0 for _ in shape)
  return pl.BlockSpec(shape, idx)


_phi = pl.pallas_call(
    _phi_kernel,
    grid=(E // _BE,),
    in_specs=[
        pl.BlockSpec((_BE, FP), lambda i: (i, 0)),
        pl.BlockSpec((_BE, FP), lambda i: (i, 0)),
        pl.BlockSpec((_BE, STATE_DIM), lambda i: (i, 0)),
        _full((FP, 64)), _full((FP, 64)), _full((STATE_DIM, 64)),
        _full((1, 64)), _full((64, 64)), _full((1, 64)),
        _full((64, 64)), _full((64, 1)),
    ],
    out_specs=pl.BlockSpec((_BE // 128, 64, 128), lambda i: (i, 0, 0)),
    out_shape=jax.ShapeDtypeStruct((NROW, 64, 128), F32),
    name="tc_phi",
)


def _gamma_kernel(ag, h, wa, wh, b1, w2, b2, w3, b3, out):
  agf = ag[...].reshape(64, _BN)
  y = lax.dot_general(agf, wa[...], (((0,), (0,)), ((), ())),
                      preferred_element_type=F32) + _dot(h[...], wh[...])
  y = jnp.maximum(y + b1[...], 0.0)
  y = jnp.maximum(_dot(y, w2[...]) + b2[...], 0.0)
  y = _dot(y, w3[...]) + b3[...]
  y = jnp.maximum(y, 0.0)
  out[...] = jnp.concatenate([y, jnp.zeros((_BN, FP - 64), F32)], axis=1)


_gamma1 = pl.pallas_call(
    _gamma_kernel,
    grid=(_GN,),
    in_specs=[
        pl.BlockSpec((64, _BN // 128, 128), lambda i: (0, i, 0)),
        pl.BlockSpec((_BN, FP), lambda i: (i, 0)),
        _full((64, 64)), _full((FP, 64)), _full((1, 64)),
        _full((64, 64)), _full((1, 64)),
        _full((64, 64)), _full((1, 64)),
    ],
    out_specs=pl.BlockSpec((_BN, FP), lambda i: (i, 0)),
    out_shape=jax.ShapeDtypeStruct((N, FP), F32),
    name="tc_gamma1",
)


def _gamma_head_kernel(ag, h, wa, wh, b1, w2, b2, w3, b3,
                       hw1, hb1, hw2, hb2, hw3, hb3, out):
  agf = ag[...].reshape(64, _BN)
  y = lax.dot_general(agf, wa[...], (((0,), (0,)), ((), ())),
                      preferred_element_type=F32) + _dot(h[...], wh[...])
  y = jnp.maximum(y + b1[...], 0.0)
  y = jnp.maximum(_dot(y, w2[...]) + b2[...], 0.0)
  y = _dot(y, w3[...]) + b3[...]
  z = jnp.maximum(_dot(y, hw1[...]) + hb1[...], 0.0)
  z = jnp.maximum(_dot(z, hw2[...]) + hb2[...], 0.0)
  out[...] = _dot(z, hw3[...]) + hb3[...]


_gamma_head = pl.pallas_call(
    _gamma_head_kernel,
    grid=(_GN,),
    in_specs=[
        pl.BlockSpec((64, _BN // 128, 128), lambda i: (0, i, 0)),
        pl.BlockSpec((_BN, FP), lambda i: (i, 0)),
        _full((64, 64)), _full((FP, 64)), _full((1, 64)),
        _full((64, 64)), _full((1, 64)),
        _full((64, 64)), _full((1, 64)),
        _full((64, 64)), _full((1, 64)),
        _full((64, 64)), _full((1, 64)),
        _full((64, 1)), _full((1, 1)),
    ],
    out_specs=pl.BlockSpec((_BN, 1), lambda i: (i, 0)),
    out_shape=jax.ShapeDtypeStruct((N, 1), F32),
    name="tc_gamma_head",
)


def _pad_rows(w):
  return jnp.concatenate(
      [w, jnp.zeros((FP - w.shape[0], w.shape[1]), F32)], axis=0)


def _split_phi_w(p, feat):
  (w1, b1), (w2, b2), (w3, b3) = p
  wd = _pad_rows(w1[:feat])
  ws = _pad_rows(w1[feat:2 * feat])
  wa = w1[2 * feat:]
  return (wd, ws, wa, b1.reshape(1, 64), w2, b2.reshape(1, 64),
          w3, b3.reshape(64, 1))


def _split_gamma_w(p):
  (w1, b1), (w2, b2), (w3, b3) = p
  wa = w1[:64]
  wh = _pad_rows(w1[64:])
  return (wa, wh, b1.reshape(1, 64), w2, b2.reshape(1, 64),
          w3, b3.reshape(1, 64))


def kernel(x, edge_attr, edge_index, params):
  src = edge_index[0]
  dst = edge_index[1]
  xp = jnp.concatenate([x, jnp.zeros((N, FP - STATE_DIM), F32)], axis=1)

  hs1, hd1 = _gather(xp, src, dst)
  msg1 = _phi(hd1, hs1, edge_attr, *_split_phi_w(params['phi1'], STATE_DIM))
  ag1 = _segmax(msg1, dst)
  h1 = _gamma1(ag1, xp, *_split_gamma_w(params['gamma1']))

  hs2, hd2 = _gather(h1, src, dst)
  msg2 = _phi(hd2, hs2, edge_attr, *_split_phi_w(params['phi2'], 64))
  ag2 = _segmax(msg2, dst)

  (hw1, hb1), (hw2, hb2), (hw3, hb3) = params['head']
  out = _gamma_head(ag2, h1, *_split_gamma_w(params['gamma2']),
                    hw1, hb1.reshape(1, 64), hw2, hb2.reshape(1, 64),
                    hw3, hb3.reshape(1, 1))
  return out.reshape(-1, NUM_AGENTS)


# speedup vs baseline: 4.6864x; 1.1745x over previous
import functools

import jax
import jax.numpy as jnp
from jax import lax
from jax.experimental import pallas as pl
from jax.experimental.pallas import tpu as pltpu
from jax.experimental.pallas import tpu_sc as plsc

N = 50000
E = 800000
STATE_DIM = 16
NUM_AGENTS = 1000
FP = 128
NROW = E // FP
NPAD = 392

NC = 2
NS = 16
NW = NC * NS

_MESH = plsc.VectorSubcoreMesh(
    core_axis_name="c", subcore_axis_name="s", num_cores=NC, num_subcores=NS)

F32 = jnp.float32


def _worker_id():
  return lax.axis_index("s") * NC + lax.axis_index("c")


def _gather_body(pd_hbm, ps_hbm, src_hbm, dst_hbm, g_hbm,
                 idx_d, idx_s, rows_v, sem, semi):
  wid = _worker_id()
  epw = E // NW
  chb = 1000
  base = wid * epw

  def fire(tbl, idx, add):
    descs = []
    for j in range(7):
      descs.append(pltpu.async_copy(
          tbl.at[idx.at[pl.ds(j * 128, 128)]],
          rows_v.at[pl.ds(j * 128, 128)], sem, add=add))
    descs.append(pltpu.async_copy(
        tbl.at[idx.at[pl.ds(896, 104)]],
        rows_v.at[pl.ds(896, 104)], sem, add=add))
    for d in descs:
      d.wait()

  def chunk(i, carry):
    off = base + i * chb
    pltpu.async_copy(src_hbm.at[pl.ds(off, chb)], idx_s, semi)
    pltpu.sync_copy(dst_hbm.at[pl.ds(off, chb)], idx_d)
    fire(pd_hbm, idx_d, False)
    pltpu.make_async_copy(src_hbm.at[pl.ds(off, chb)], idx_s, semi).wait()
    fire(ps_hbm, idx_s, True)
    pltpu.sync_copy(rows_v, g_hbm.at[pl.ds(off, chb)])
    return carry
  lax.fori_loop(0, epw // chb, chunk, 0)


_gather = pl.kernel(
    _gather_body,
    out_type=jax.ShapeDtypeStruct((E, FP), F32),
    mesh=_MESH,
    scratch_types=[
        pltpu.VMEM((1000,), jnp.int32),
        pltpu.VMEM((1000,), jnp.int32),
        pltpu.VMEM((1000, FP), F32),
        pltpu.SemaphoreType.DMA,
        pltpu.SemaphoreType.DMA,
    ],
    compiler_params=pltpu.CompilerParams(use_tc_tiling_on_sc=False),
    name="sc_gather",
)


_SEG_CH = 3200
_SEG_ROWS = _SEG_CH // 128
_NCHUNK = E // _SEG_CH
_NEG = -3.0e38


def _segmax_body(msgT_hbm, dst_hbm, aggrT_hbm, dstb, mc, acc,
                 sd0, sd1, sm0, sm1):
  wid = _worker_id()
  f0 = wid * 2
  dsems = (sd0, sd1)
  msems = (sm0, sm1)

  def init_row(r, carry):
    for f in range(2):
      for k in range(8):
        acc[f, r, pl.ds(k * 16, 16)] = jnp.full((16,), _NEG, F32)
    return carry
  lax.fori_loop(0, NPAD, init_row, 0)

  def issue(c, buf):
    pltpu.async_copy(dst_hbm.at[pl.ds(c * _SEG_CH, _SEG_CH)],
                     dstb.at[buf], dsems[buf])
    pltpu.async_copy(
        msgT_hbm.at[pl.ds(c * _SEG_ROWS, _SEG_ROWS), pl.ds(f0, 2)],
        mc.at[buf], msems[buf])

  def wait(buf):
    pltpu.make_async_copy(dst_hbm.at[pl.ds(0, _SEG_CH)],
                          dstb.at[buf], dsems[buf]).wait()
    pltpu.make_async_copy(msgT_hbm.at[pl.ds(0, _SEG_ROWS), pl.ds(0, 2)],
                          mc.at[buf], msems[buf]).wait()

  issue(0, 0)
  issue(1, 1)

  zeros16 = jnp.zeros((16,), jnp.int32)
  ones16 = jnp.ones((16,), jnp.int32)
  false16 = jnp.zeros((16,), jnp.bool_)

  def window_indices(buf, rw, sub):
    d16 = dstb[buf, pl.ds(rw * 128 + sub * 16, 16)]
    dhi = d16 >> 7
    dlo = d16 & 127
    return [zeros16, dhi, dlo], [ones16, dhi, dlo]

  def safe_update(buf, rw, sub):
    i0, i1 = window_indices(buf, rw, sub)
    m0 = mc[buf, rw, 0, pl.ds(sub * 16, 16)]
    m1 = mc[buf, rw, 1, pl.ds(sub * 16, 16)]
    n0 = jnp.maximum(plsc.load_gather(acc, i0), m0)
    n1 = jnp.maximum(plsc.load_gather(acc, i1), m1)

    def cond(st):
      return st[2] > 0

    def fbody(st):
      l0, l1, _ = st
      plsc.store_scatter(acc, i0, n0, mask=l0)
      plsc.store_scatter(acc, i1, n1, mask=l1)
      l0n = (plsc.load_gather(acc, i0) < n0) & l0
      l1n = (plsc.load_gather(acc, i1) < n1) & l1
      return (l0n, l1n, plsc.all_reduce_population_count(l0n | l1n)[0])
    lax.while_loop(cond, fbody,
                   (~false16, ~false16, jnp.int32(16)))

  def row_body(buf):
    def body(rw, carry):
      lostacc = false16
      for sub in range(8):
        i0, i1 = window_indices(buf, rw, sub)
        m0 = mc[buf, rw, 0, pl.ds(sub * 16, 16)]
        m1 = mc[buf, rw, 1, pl.ds(sub * 16, 16)]
        n0 = jnp.maximum(plsc.load_gather(acc, i0), m0)
        n1 = jnp.maximum(plsc.load_gather(acc, i1), m1)
        plsc.store_scatter(acc, i0, n0)
        plsc.store_scatter(acc, i1, n1)
        lost0 = plsc.load_gather(acc, i0) < n0
        lost1 = plsc.load_gather(acc, i1) < n1
        lostacc = lostacc | lost0 | lost1
      nl = plsc.all_reduce_population_count(lostacc)[0]

      @pl.when(nl > 0)
      def _fix():
        for sub in range(8):
          safe_update(buf, rw, sub)
      return carry
    return body

  def chunk_pair(c2, carry):
    for buf in (0, 1):
      c = c2 * 2 + buf
      wait(buf)
      lax.fori_loop(0, _SEG_ROWS, row_body(buf), 0)

      @pl.when(c + 2 < _NCHUNK)
      def _pref():
        issue(c + 2, buf)
    return carry
  lax.fori_loop(0, _NCHUNK // 2, chunk_pair, 0)

  def fin_row(r, carry):
    for f in range(2):
      for k in range(8):
        sl = pl.ds(k * 16, 16)
        v = acc[f, r, sl]
        acc[f, r, sl] = jnp.where(v > -1.0e38, v, 0.0)
    return carry
  lax.fori_loop(0, NPAD, fin_row, 0)

  pltpu.sync_copy(acc, aggrT_hbm.at[pl.ds(f0, 2)])


_segmax = pl.kernel(
    _segmax_body,
    out_type=jax.ShapeDtypeStruct((64, NPAD, 128), F32),
    mesh=_MESH,
    scratch_types=[
        pltpu.VMEM((2, _SEG_CH), jnp.int32),
        pltpu.VMEM((2, _SEG_ROWS, 2, 128), F32),
        pltpu.VMEM((2, NPAD, 128), F32),
        pltpu.SemaphoreType.DMA,
        pltpu.SemaphoreType.DMA,
        pltpu.SemaphoreType.DMA,
        pltpu.SemaphoreType.DMA,
    ],
    compiler_params=pltpu.CompilerParams(
        use_tc_tiling_on_sc=False, needs_layout_passes=False),
    name="sc_segmax",
)


_BE = 3200
_BN = 2048
_GN = -(-N // _BN)


def _dot(a, b):
  return jnp.dot(a, b, preferred_element_type=F32)


def _proj_kernel(h, wd, ws, outd, outs):
  z = jnp.zeros((_BN, 64), F32)
  outd[...] = jnp.concatenate([_dot(h[...], wd[...]), z], axis=1)
  outs[...] = jnp.concatenate([z, _dot(h[...], ws[...])], axis=1)


def _make_proj(feat):
  return pl.pallas_call(
      _proj_kernel,
      grid=(_GN,),
      in_specs=[
          pl.BlockSpec((_BN, feat), lambda i: (i, 0)),
          _full((feat, 64)), _full((feat, 64)),
      ],
      out_specs=[pl.BlockSpec((_BN, FP), lambda i: (i, 0)),
                 pl.BlockSpec((_BN, FP), lambda i: (i, 0))],
      out_shape=[jax.ShapeDtypeStruct((N, FP), F32),
                 jax.ShapeDtypeStruct((N, FP), F32)],
      name=f"tc_proj_f{feat}",
  )


def _phi_kernel(g, at, wa, b1, w2, b2, w3, b3, out):
  gg = g[...]
  y = gg[:, :64] + gg[:, 64:] + _dot(at[...], wa[...])
  y = jnp.maximum(y + b1[...], 0.0)
  y = jnp.maximum(_dot(y, w2[...]) + b2[...], 0.0)
  yt = lax.dot_general(
      w3[...], y, (((0,), (1,)), ((), ())),
      preferred_element_type=F32) + b3[...]
  for r in range(_BE // 128):
    out[r] = yt[:, r * 128:(r + 1) * 128]


def _full(shape):
  idx = lambda i: tuple(0 for _ in shape)
  return pl.BlockSpec(shape, idx)


_phi = pl.pallas_call(
    _phi_kernel,
    grid=(E // _BE,),
    in_specs=[
        pl.BlockSpec((_BE, FP), lambda i: (i, 0)),
        pl.BlockSpec((_BE, STATE_DIM), lambda i: (i, 0)),
        _full((STATE_DIM, 64)),
        _full((1, 64)), _full((64, 64)), _full((1, 64)),
        _full((64, 64)), _full((64, 1)),
    ],
    out_specs=pl.BlockSpec((_BE // 128, 64, 128), lambda i: (i, 0, 0)),
    out_shape=jax.ShapeDtypeStruct((NROW, 64, 128), F32),
    name="tc_phi",
)


def _gamma_kernel(ag, h, wa, wh, b1, w2, b2, w3, b3, out):
  agf = ag[...].reshape(64, _BN)
  y = lax.dot_general(agf, wa[...], (((0,), (0,)), ((), ())),
                      preferred_element_type=F32) + _dot(h[...], wh[...])
  y = jnp.maximum(y + b1[...], 0.0)
  y = jnp.maximum(_dot(y, w2[...]) + b2[...], 0.0)
  y = _dot(y, w3[...]) + b3[...]
  out[...] = jnp.maximum(y, 0.0)


_gamma1 = pl.pallas_call(
    _gamma_kernel,
    grid=(_GN,),
    in_specs=[
        pl.BlockSpec((64, _BN // 128, 128), lambda i: (0, i, 0)),
        pl.BlockSpec((_BN, STATE_DIM), lambda i: (i, 0)),
        _full((64, 64)), _full((STATE_DIM, 64)), _full((1, 64)),
        _full((64, 64)), _full((1, 64)),
        _full((64, 64)), _full((1, 64)),
    ],
    out_specs=pl.BlockSpec((_BN, 64), lambda i: (i, 0)),
    out_shape=jax.ShapeDtypeStruct((N, 64), F32),
    name="tc_gamma1",
)


def _gamma_head_kernel(ag, h, wa, wh, b1, w2, b2, w3, b3,
                       hw1, hb1, hw2, hb2, hw3, hb3, out):
  agf = ag[...].reshape(64, _BN)
  y = lax.dot_general(agf, wa[...], (((0,), (0,)), ((), ())),
                      preferred_element_type=F32) + _dot(h[...], wh[...])
  y = jnp.maximum(y + b1[...], 0.0)
  y = jnp.maximum(_dot(y, w2[...]) + b2[...], 0.0)
  y = _dot(y, w3[...]) + b3[...]
  z = jnp.maximum(_dot(y, hw1[...]) + hb1[...], 0.0)
  z = jnp.maximum(_dot(z, hw2[...]) + hb2[...], 0.0)
  out[...] = _dot(z, hw3[...]) + hb3[...]


_gamma_head = pl.pallas_call(
    _gamma_head_kernel,
    grid=(_GN,),
    in_specs=[
        pl.BlockSpec((64, _BN // 128, 128), lambda i: (0, i, 0)),
        pl.BlockSpec((_BN, 64), lambda i: (i, 0)),
        _full((64, 64)), _full((64, 64)), _full((1, 64)),
        _full((64, 64)), _full((1, 64)),
        _full((64, 64)), _full((1, 64)),
        _full((64, 64)), _full((1, 64)),
        _full((64, 64)), _full((1, 64)),
        _full((64, 1)), _full((1, 1)),
    ],
    out_specs=pl.BlockSpec((_BN, 1), lambda i: (i, 0)),
    out_shape=jax.ShapeDtypeStruct((N, 1), F32),
    name="tc_gamma_head",
)


_proj16 = _make_proj(STATE_DIM)
_proj64 = _make_proj(64)


def _split_phi_w(p, feat):
  (w1, b1), (w2, b2), (w3, b3) = p
  return ((w1[:feat], w1[feat:2 * feat]),
          (w1[2 * feat:], b1.reshape(1, 64), w2, b2.reshape(1, 64),
           w3, b3.reshape(64, 1)))


def _split_gamma_w(p):
  (w1, b1), (w2, b2), (w3, b3) = p
  return (w1[:64], w1[64:], b1.reshape(1, 64), w2, b2.reshape(1, 64),
          w3, b3.reshape(1, 64))


def kernel(x, edge_attr, edge_index, params):
  src = edge_index[0]
  dst = edge_index[1]

  (wd1, ws1), phi1_rest = _split_phi_w(params['phi1'], STATE_DIM)
  pd1, ps1 = _proj16(x, wd1, ws1)
  g1 = _gather(pd1, ps1, src, dst)
  msg1 = _phi(g1, edge_attr, *phi1_rest)
  ag1 = _segmax(msg1, dst)
  h1 = _gamma1(ag1, x, *_split_gamma_w(params['gamma1']))

  (wd2, ws2), phi2_rest = _split_phi_w(params['phi2'], 64)
  pd2, ps2 = _proj64(h1, wd2, ws2)
  g2 = _gather(pd2, ps2, src, dst)
  msg2 = _phi(g2, edge_attr, *phi2_rest)
  ag2 = _segmax(msg2, dst)

  (hw1, hb1), (hw2, hb2), (hw3, hb3) = params['head']
  out = _gamma_head(ag2, h1, *_split_gamma_w(params['gamma2']),
                    hw1, hb1.reshape(1, 64), hw2, hb2.reshape(1, 64),
                    hw3, hb3.reshape(1, 1))
  return out.reshape(-1, NUM_AGENTS)


# speedup vs baseline: 4.7795x; 1.0199x over previous
import functools

import jax
import jax.numpy as jnp
from jax import lax
from jax.experimental import pallas as pl
from jax.experimental.pallas import tpu as pltpu
from jax.experimental.pallas import tpu_sc as plsc

N = 50000
E = 800000
STATE_DIM = 16
NUM_AGENTS = 1000
FP = 128
NROW = E // FP
NPAD = 392

NC = 2
NS = 16
NW = NC * NS

_MESH = plsc.VectorSubcoreMesh(
    core_axis_name="c", subcore_axis_name="s", num_cores=NC, num_subcores=NS)

F32 = jnp.float32


def _worker_id():
  return lax.axis_index("s") * NC + lax.axis_index("c")


_GCH = 200
_GSUB = ((0, 128), (128, 72))


def _gather_body(pd_hbm, ps_hbm, src_hbm, dst_hbm, g_hbm,
                 id0, id1, is0, is1, rows_v, sg0, sg1, si0, si1, so0, so1):
  wid = _worker_id()
  epw = E // NW
  base = wid * epw
  nch = epw // _GCH
  idx_d = (id0, id1)
  idx_s = (is0, is1)
  sgs = (sg0, sg1)
  sis = (si0, si1)
  sos = (so0, so1)

  def issue_idx(c, slot):
    off = base + c * _GCH
    pltpu.async_copy(dst_hbm.at[pl.ds(off, _GCH)], idx_d[slot], sis[slot])
    pltpu.async_copy(src_hbm.at[pl.ds(off, _GCH)], idx_s[slot], sis[slot])

  def wait_idx(slot):
    pltpu.make_async_copy(dst_hbm.at[pl.ds(0, _GCH)], idx_d[slot],
                          sis[slot]).wait()
    pltpu.make_async_copy(src_hbm.at[pl.ds(0, _GCH)], idx_s[slot],
                          sis[slot]).wait()

  def fire(tbl, idxs, slot, add):
    descs = []
    for (o, l) in _GSUB:
      descs.append(pltpu.async_copy(
          tbl.at[idxs[slot].at[pl.ds(o, l)]],
          rows_v.at[slot].at[pl.ds(o, l)], sgs[slot], add=add))
    for d in descs:
      d.wait()

  issue_idx(0, 0)
  issue_idx(1, 1)

  def process(c, slot):
    @pl.when(c >= 2)
    def _wo():
      pltpu.make_async_copy(rows_v.at[slot],
                            g_hbm.at[pl.ds(0, _GCH)], sos[slot]).wait()
    wait_idx(slot)
    fire(pd_hbm, idx_d, slot, False)
    fire(ps_hbm, idx_s, slot, True)
    pltpu.async_copy(rows_v.at[slot],
                     g_hbm.at[pl.ds(base + c * _GCH, _GCH)], sos[slot])

    @pl.when(c + 2 < nch)
    def _pref():
      issue_idx(c + 2, slot)

  def chunk_pair(c2, carry):
    for slot in (0, 1):
      process(c2 * 2 + slot, slot)
    return carry
  lax.fori_loop(0, nch // 2, chunk_pair, 0)
  if nch % 2:
    process(nch - 1, 0)

  for slot in (0, 1):
    pltpu.make_async_copy(rows_v.at[slot],
                          g_hbm.at[pl.ds(0, _GCH)], sos[slot]).wait()


_gather = pl.kernel(
    _gather_body,
    out_type=jax.ShapeDtypeStruct((E, FP), F32),
    mesh=_MESH,
    scratch_types=[
        pltpu.VMEM((_GCH,), jnp.int32),
        pltpu.VMEM((_GCH,), jnp.int32),
        pltpu.VMEM((_GCH,), jnp.int32),
        pltpu.VMEM((_GCH,), jnp.int32),
        pltpu.VMEM((2, _GCH, FP), F32),
        pltpu.SemaphoreType.DMA,
        pltpu.SemaphoreType.DMA,
        pltpu.SemaphoreType.DMA,
        pltpu.SemaphoreType.DMA,
        pltpu.SemaphoreType.DMA,
        pltpu.SemaphoreType.DMA,
    ],
    compiler_params=pltpu.CompilerParams(use_tc_tiling_on_sc=False),
    name="sc_gather",
)


_SEG_CH = 3200
_SEG_ROWS = _SEG_CH // 128
_NCHUNK = E // _SEG_CH
_NEG = -3.0e38


def _segmax_body(msgT_hbm, dst_hbm, aggrT_hbm, dstb, mc, acc,
                 sd0, sd1, sm0, sm1):
  wid = _worker_id()
  f0 = wid * 2
  dsems = (sd0, sd1)
  msems = (sm0, sm1)

  def init_row(r, carry):
    for f in range(2):
      for k in range(8):
        acc[f, r, pl.ds(k * 16, 16)] = jnp.full((16,), _NEG, F32)
    return carry
  lax.fori_loop(0, NPAD, init_row, 0)

  def issue(c, buf):
    pltpu.async_copy(dst_hbm.at[pl.ds(c * _SEG_CH, _SEG_CH)],
                     dstb.at[buf], dsems[buf])
    pltpu.async_copy(
        msgT_hbm.at[pl.ds(c * _SEG_ROWS, _SEG_ROWS), pl.ds(f0, 2)],
        mc.at[buf], msems[buf])

  def wait(buf):
    pltpu.make_async_copy(dst_hbm.at[pl.ds(0, _SEG_CH)],
                          dstb.at[buf], dsems[buf]).wait()
    pltpu.make_async_copy(msgT_hbm.at[pl.ds(0, _SEG_ROWS), pl.ds(0, 2)],
                          mc.at[buf], msems[buf]).wait()

  issue(0, 0)
  issue(1, 1)

  zeros16 = jnp.zeros((16,), jnp.int32)
  ones16 = jnp.ones((16,), jnp.int32)
  false16 = jnp.zeros((16,), jnp.bool_)

  def window_indices(buf, rw, sub):
    d16 = dstb[buf, pl.ds(rw * 128 + sub * 16, 16)]
    dhi = d16 >> 7
    dlo = d16 & 127
    return [zeros16, dhi, dlo], [ones16, dhi, dlo]

  def safe_update(buf, rw, sub):
    i0, i1 = window_indices(buf, rw, sub)
    m0 = mc[buf, rw, 0, pl.ds(sub * 16, 16)]
    m1 = mc[buf, rw, 1, pl.ds(sub * 16, 16)]
    n0 = jnp.maximum(plsc.load_gather(acc, i0), m0)
    n1 = jnp.maximum(plsc.load_gather(acc, i1), m1)

    def cond(st):
      return st[2] > 0

    def fbody(st):
      l0, l1, _ = st
      plsc.store_scatter(acc, i0, n0, mask=l0)
      plsc.store_scatter(acc, i1, n1, mask=l1)
      l0n = (plsc.load_gather(acc, i0) < n0) & l0
      l1n = (plsc.load_gather(acc, i1) < n1) & l1
      return (l0n, l1n, plsc.all_reduce_population_count(l0n | l1n)[0])
    lax.while_loop(cond, fbody,
                   (~false16, ~false16, jnp.int32(16)))

  def row_body(buf):
    def body(rw, carry):
      lostacc = false16
      for sub in range(8):
        i0, i1 = window_indices(buf, rw, sub)
        m0 = mc[buf, rw, 0, pl.ds(sub * 16, 16)]
        m1 = mc[buf, rw, 1, pl.ds(sub * 16, 16)]
        n0 = jnp.maximum(plsc.load_gather(acc, i0), m0)
        n1 = jnp.maximum(plsc.load_gather(acc, i1), m1)
        plsc.store_scatter(acc, i0, n0)
        plsc.store_scatter(acc, i1, n1)
        lost0 = plsc.load_gather(acc, i0) < n0
        lost1 = plsc.load_gather(acc, i1) < n1
        lostacc = lostacc | lost0 | lost1
      nl = plsc.all_reduce_population_count(lostacc)[0]

      @pl.when(nl > 0)
      def _fix():
        for sub in range(8):
          safe_update(buf, rw, sub)
      return carry
    return body

  def chunk_pair(c2, carry):
    for buf in (0, 1):
      c = c2 * 2 + buf
      wait(buf)
      lax.fori_loop(0, _SEG_ROWS, row_body(buf), 0)

      @pl.when(c + 2 < _NCHUNK)
      def _pref():
        issue(c + 2, buf)
    return carry
  lax.fori_loop(0, _NCHUNK // 2, chunk_pair, 0)

  def fin_row(r, carry):
    for f in range(2):
      for k in range(8):
        sl = pl.ds(k * 16, 16)
        v = acc[f, r, sl]
        acc[f, r, sl] = jnp.where(v > -1.0e38, v, 0.0)
    return carry
  lax.fori_loop(0, NPAD, fin_row, 0)

  pltpu.sync_copy(acc, aggrT_hbm.at[pl.ds(f0, 2)])


_segmax = pl.kernel(
    _segmax_body,
    out_type=jax.ShapeDtypeStruct((64, NPAD, 128), F32),
    mesh=_MESH,
    scratch_types=[
        pltpu.VMEM((2, _SEG_CH), jnp.int32),
        pltpu.VMEM((2, _SEG_ROWS, 2, 128), F32),
        pltpu.VMEM((2, NPAD, 128), F32),
        pltpu.SemaphoreType.DMA,
        pltpu.SemaphoreType.DMA,
        pltpu.SemaphoreType.DMA,
        pltpu.SemaphoreType.DMA,
    ],
    compiler_params=pltpu.CompilerParams(
        use_tc_tiling_on_sc=False, needs_layout_passes=False),
    name="sc_segmax",
)


_BE = 6400
_BN = 2048
_GN = -(-N // _BN)


def _dot(a, b):
  return jnp.dot(a, b, preferred_element_type=F32)


def _proj_kernel(h, wd, ws, outd, outs):
  z = jnp.zeros((_BN, 64), F32)
  outd[...] = jnp.concatenate([_dot(h[...], wd[...]), z], axis=1)
  outs[...] = jnp.concatenate([z, _dot(h[...], ws[...])], axis=1)


def _make_proj(feat):
  return pl.pallas_call(
      _proj_kernel,
      grid=(_GN,),
      in_specs=[
          pl.BlockSpec((_BN, feat), lambda i: (i, 0)),
          _full((feat, 64)), _full((feat, 64)),
      ],
      out_specs=[pl.BlockSpec((_BN, FP), lambda i: (i, 0)),
                 pl.BlockSpec((_BN, FP), lambda i: (i, 0))],
      out_shape=[jax.ShapeDtypeStruct((N, FP), F32),
                 jax.ShapeDtypeStruct((N, FP), F32)],
      name=f"tc_proj_f{feat}",
  )


def _phi_kernel(g, at, wa, b1, w2, b2, w3, b3, out):
  gg = g[...]
  y = gg[:, :64] + gg[:, 64:] + _dot(at[...], wa[...])
  y = jnp.maximum(y + b1[...], 0.0)
  y = jnp.maximum(_dot(y, w2[...]) + b2[...], 0.0)
  yt = lax.dot_general(
      w3[...], y, (((0,), (1,)), ((), ())),
      preferred_element_type=F32) + b3[...]
  for r in range(_BE // 128):
    out[r] = yt[:, r * 128:(r + 1) * 128]


def _full(shape):
  idx = lambda i: tuple(0 for _ in shape)
  return pl.BlockSpec(shape, idx)


_phi = pl.pallas_call(
    _phi_kernel,
    grid=(E // _BE,),
    in_specs=[
        pl.BlockSpec((_BE, FP), lambda i: (i, 0)),
        pl.BlockSpec((_BE, STATE_DIM), lambda i: (i, 0)),
        _full((STATE_DIM, 64)),
        _full((1, 64)), _full((64, 64)), _full((1, 64)),
        _full((64, 64)), _full((64, 1)),
    ],
    out_specs=pl.BlockSpec((_BE // 128, 64, 128), lambda i: (i, 0, 0)),
    out_shape=jax.ShapeDtypeStruct((NROW, 64, 128), F32),
    name="tc_phi",
)


def _gamma_kernel(ag, h, wa, wh, b1, w2, b2, w3, b3, out):
  agf = ag[...].reshape(64, _BN)
  y = lax.dot_general(agf, wa[...], (((0,), (0,)), ((), ())),
                      preferred_element_type=F32) + _dot(h[...], wh[...])
  y = jnp.maximum(y + b1[...], 0.0)
  y = jnp.maximum(_dot(y, w2[...]) + b2[...], 0.0)
  y = _dot(y, w3[...]) + b3[...]
  out[...] = jnp.maximum(y, 0.0)


_gamma1 = pl.pallas_call(
    _gamma_kernel,
    grid=(_GN,),
    in_specs=[
        pl.BlockSpec((64, _BN // 128, 128), lambda i: (0, i, 0)),
        pl.BlockSpec((_BN, STATE_DIM), lambda i: (i, 0)),
        _full((64, 64)), _full((STATE_DIM, 64)), _full((1, 64)),
        _full((64, 64)), _full((1, 64)),
        _full((64, 64)), _full((1, 64)),
    ],
    out_specs=pl.BlockSpec((_BN, 64), lambda i: (i, 0)),
    out_shape=jax.ShapeDtypeStruct((N, 64), F32),
    name="tc_gamma1",
)


def _gamma_head_kernel(ag, h, wa, wh, b1, w2, b2, w3, b3,
                       hw1, hb1, hw2, hb2, hw3, hb3, out):
  agf = ag[...].reshape(64, _BN)
  y = lax.dot_general(agf, wa[...], (((0,), (0,)), ((), ())),
                      preferred_element_type=F32) + _dot(h[...], wh[...])
  y = jnp.maximum(y + b1[...], 0.0)
  y = jnp.maximum(_dot(y, w2[...]) + b2[...], 0.0)
  y = _dot(y, w3[...]) + b3[...]
  z = jnp.maximum(_dot(y, hw1[...]) + hb1[...], 0.0)
  z = jnp.maximum(_dot(z, hw2[...]) + hb2[...], 0.0)
  out[...] = _dot(z, hw3[...]) + hb3[...]


_gamma_head = pl.pallas_call(
    _gamma_head_kernel,
    grid=(_GN,),
    in_specs=[
        pl.BlockSpec((64, _BN // 128, 128), lambda i: (0, i, 0)),
        pl.BlockSpec((_BN, 64), lambda i: (i, 0)),
        _full((64, 64)), _full((64, 64)), _full((1, 64)),
        _full((64, 64)), _full((1, 64)),
        _full((64, 64)), _full((1, 64)),
        _full((64, 64)), _full((1, 64)),
        _full((64, 64)), _full((1, 64)),
        _full((64, 1)), _full((1, 1)),
    ],
    out_specs=pl.BlockSpec((_BN, 1), lambda i: (i, 0)),
    out_shape=jax.ShapeDtypeStruct((N, 1), F32),
    name="tc_gamma_head",
)


_proj16 = _make_proj(STATE_DIM)
_proj64 = _make_proj(64)


def _split_phi_w(p, feat):
  (w1, b1), (w2, b2), (w3, b3) = p
  return ((w1[:feat], w1[feat:2 * feat]),
          (w1[2 * feat:], b1.reshape(1, 64), w2, b2.reshape(1, 64),
           w3, b3.reshape(64, 1)))


def _split_gamma_w(p):
  (w1, b1), (w2, b2), (w3, b3) = p
  return (w1[:64], w1[64:], b1.reshape(1, 64), w2, b2.reshape(1, 64),
          w3, b3.reshape(1, 64))


def kernel(x, edge_attr, edge_index, params):
  src = edge_index[0]
  dst = edge_index[1]

  (wd1, ws1), phi1_rest = _split_phi_w(params['phi1'], STATE_DIM)
  pd1, ps1 = _proj16(x, wd1, ws1)
  g1 = _gather(pd1, ps1, src, dst)
  msg1 = _phi(g1, edge_attr, *phi1_rest)
  ag1 = _segmax(msg1, dst)
  h1 = _gamma1(ag1, x, *_split_gamma_w(params['gamma1']))

  (wd2, ws2), phi2_rest = _split_phi_w(params['phi2'], 64)
  pd2, ps2 = _proj64(h1, wd2, ws2)
  g2 = _gather(pd2, ps2, src, dst)
  msg2 = _phi(g2, edge_attr, *phi2_rest)
  ag2 = _segmax(msg2, dst)

  (hw1, hb1), (hw2, hb2), (hw3, hb3) = params['head']
  out = _gamma_head(ag2, h1, *_split_gamma_w(params['gamma2']),
                    hw1, hb1.reshape(1, 64), hw2, hb2.reshape(1, 64),
                    hw3, hb3.reshape(1, 1))
  return out.reshape(-1, NUM_AGENTS)
